# baseline (device time: 22846 ns/iter reference)
import jax
import jax.numpy as jnp
from jax import lax
from jax.experimental import pallas as pl
from jax.experimental.pallas import tpu as pltpu

N_DEV = 16
H = 4


def kernel(A, B):
    m, k = A.shape
    _, n = B.shape
    ch = m // N_DEV
    hr = ch // H

    def body(a_ref, b_ref, out_ref, part_ref, rs_ref, bc_ref,
             rs_s, rs_r, ag_s, ag_r):
        my = lax.axis_index("i")

        barrier = pltpu.get_barrier_semaphore()
        for s in range(1, N_DEV):
            dst = lax.rem(my + s, N_DEV)
            pl.semaphore_signal(
                barrier, inc=1, device_id=(dst,),
                device_id_type=pl.DeviceIdType.MESH,
            )
        a = a_ref[...].astype(jnp.bfloat16)
        b = b_ref[...].astype(jnp.bfloat16)
        part_ref[...] = jnp.dot(
            a, b, preferred_element_type=jnp.float32
        ).astype(jnp.bfloat16)
        pl.semaphore_wait(barrier, N_DEV - 1)

        rs = {}
        for h in range(H):
            for s in range(1, N_DEV):
                dst = lax.rem(my + s, N_DEV)
                rdma = pltpu.make_async_remote_copy(
                    src_ref=part_ref.at[pl.ds(dst * ch + h * hr, hr), :],
                    dst_ref=rs_ref.at[h, s],
                    send_sem=rs_s.at[h, s],
                    recv_sem=rs_r.at[h, s],
                    device_id=(dst,),
                    device_id_type=pl.DeviceIdType.MESH,
                )
                rdma.start()
                rs[h, s] = rdma

        ag = {}
        for h in range(H):
            acc = part_ref[
                pl.ds(my * ch + h * hr, hr), :
            ].astype(jnp.float32)
            for s in range(1, N_DEV):
                rs[h, s].wait_recv()
                acc = acc + rs_ref[h, s].astype(jnp.float32)
            final = jnp.maximum(acc, 0.0).astype(jnp.bfloat16)
            bc_ref[h] = final
            out_ref[pl.ds(my * ch + h * hr, hr), :] = final
            for s in range(1, N_DEV):
                dst = lax.rem(my + s, N_DEV)
                rdma = pltpu.make_async_remote_copy(
                    src_ref=bc_ref.at[h],
                    dst_ref=out_ref.at[pl.ds(my * ch + h * hr, hr), :],
                    send_sem=ag_s.at[h, s],
                    recv_sem=ag_r.at[h, s],
                    device_id=(dst,),
                    device_id_type=pl.DeviceIdType.MESH,
                )
                rdma.start()
                ag[h, s] = rdma

        for h in range(H):
            for s in range(1, N_DEV):
                ag[h, s].wait_recv()
        for rdma in list(rs.values()) + list(ag.values()):
            rdma.wait_send()

    return pl.pallas_call(
        body,
        out_shape=jax.ShapeDtypeStruct((m, n), jnp.bfloat16),
        in_specs=[
            pl.BlockSpec(memory_space=pltpu.VMEM),
            pl.BlockSpec(memory_space=pltpu.VMEM),
        ],
        out_specs=pl.BlockSpec(memory_space=pltpu.VMEM),
        scratch_shapes=[
            pltpu.VMEM((m, n), jnp.bfloat16),
            pltpu.VMEM((H, N_DEV, hr, n), jnp.bfloat16),
            pltpu.VMEM((H, hr, n), jnp.bfloat16),
            pltpu.SemaphoreType.DMA((H, N_DEV)),
            pltpu.SemaphoreType.DMA((H, N_DEV)),
            pltpu.SemaphoreType.DMA((H, N_DEV)),
            pltpu.SemaphoreType.DMA((H, N_DEV)),
        ],
        compiler_params=pltpu.CompilerParams(collective_id=0),
    )(A, B)


# device time: 22094 ns/iter; 1.0340x vs baseline; 1.0340x over previous
import jax
import jax.numpy as jnp
from jax import lax
from jax.experimental import pallas as pl
from jax.experimental.pallas import tpu as pltpu

N_DEV = 16
H = 2


def kernel(A, B):
    m, k = A.shape
    _, n = B.shape
    ch = m // N_DEV
    hr = ch // H

    def body(a_ref, b_ref, out_ref, part_ref, rs_ref, bc_ref,
             cred, rs_s, rs_r, ag_s, ag_r):
        my = lax.axis_index("i")

        barrier = pltpu.get_barrier_semaphore()
        pl.semaphore_signal(
            barrier, inc=1, device_id=(my,),
            device_id_type=pl.DeviceIdType.MESH,
        )

        for s in range(1, N_DEV):
            dst = lax.rem(my + s, N_DEV)
            pl.semaphore_signal(
                cred.at[N_DEV - s], inc=1, device_id=(dst,),
                device_id_type=pl.DeviceIdType.MESH,
            )

        a = a_ref[...].astype(jnp.bfloat16)
        b = b_ref[...].astype(jnp.bfloat16)
        part_ref[...] = jnp.dot(
            a, b, preferred_element_type=jnp.float32
        ).astype(jnp.bfloat16)
        pl.semaphore_wait(barrier, 1)

        rs = {}
        for h in range(H):
            for s in range(1, N_DEV):
                dst = lax.rem(my + s, N_DEV)
                if h == 0:
                    pl.semaphore_wait(cred.at[s], 1)
                rdma = pltpu.make_async_remote_copy(
                    src_ref=part_ref.at[pl.ds(dst * ch + h * hr, hr), :],
                    dst_ref=rs_ref.at[h, s],
                    send_sem=rs_s.at[h, s],
                    recv_sem=rs_r.at[h, s],
                    device_id=(dst,),
                    device_id_type=pl.DeviceIdType.MESH,
                )
                rdma.start()
                rs[h, s] = rdma

        ag = {}
        for h in range(H):
            acc = part_ref[
                pl.ds(my * ch + h * hr, hr), :
            ].astype(jnp.float32)
            for s in range(1, N_DEV):
                rs[h, s].wait_recv()
                acc = acc + rs_ref[h, s].astype(jnp.float32)
            bc_ref[h] = jnp.maximum(acc, 0.0).astype(jnp.bfloat16)
            for s in range(1, N_DEV):
                dst = lax.rem(my + s, N_DEV)
                rdma = pltpu.make_async_remote_copy(
                    src_ref=bc_ref.at[h],
                    dst_ref=out_ref.at[pl.ds(my * ch + h * hr, hr), :],
                    send_sem=ag_s.at[h, s],
                    recv_sem=ag_r.at[h, s],
                    device_id=(dst,),
                    device_id_type=pl.DeviceIdType.MESH,
                )
                rdma.start()
                ag[h, s] = rdma
            out_ref[pl.ds(my * ch + h * hr, hr), :] = bc_ref[h]

        for h in range(H):
            for s in range(1, N_DEV):
                ag[h, s].wait_recv()
        for rdma in list(rs.values()) + list(ag.values()):
            rdma.wait_send()

    return pl.pallas_call(
        body,
        out_shape=jax.ShapeDtypeStruct((m, n), jnp.bfloat16),
        in_specs=[
            pl.BlockSpec(memory_space=pltpu.VMEM),
            pl.BlockSpec(memory_space=pltpu.VMEM),
        ],
        out_specs=pl.BlockSpec(memory_space=pltpu.VMEM),
        scratch_shapes=[
            pltpu.VMEM((m, n), jnp.bfloat16),
            pltpu.VMEM((H, N_DEV, hr, n), jnp.bfloat16),
            pltpu.VMEM((H, hr, n), jnp.bfloat16),
            pltpu.SemaphoreType.REGULAR((N_DEV,)),
            pltpu.SemaphoreType.DMA((H, N_DEV)),
            pltpu.SemaphoreType.DMA((H, N_DEV)),
            pltpu.SemaphoreType.DMA((H, N_DEV)),
            pltpu.SemaphoreType.DMA((H, N_DEV)),
        ],
        compiler_params=pltpu.CompilerParams(collective_id=0),
    )(A, B)
